# baseline (speedup 1.0000x reference)
import functools

import jax
import jax.numpy as jnp
from jax import lax
from jax.experimental import pallas as pl
from jax.experimental.pallas import tpu as pltpu
from jax.experimental.pallas import tpu_sc as plsc

L = 16
B = 128


def _make_sc_call(n, dim, ninc):
    info = plsc.get_sparse_core_info()
    nc, ns = info.num_cores, info.num_subcores
    nw = nc * ns
    nb = n // B
    assert n % (B * nw) == 0
    blocks_w = nb // nw
    cb = min(8, blocks_w)
    assert blocks_w % (2 * cb) == 0
    n_super = blocks_w // (2 * cb)
    jpb = B // L
    jpb_bits = jpb.bit_length() - 1
    assert jpb == 1 << jpb_bits and B == 128

    mesh = plsc.VectorSubcoreMesh(core_axis_name="c", subcore_axis_name="s")

    @functools.partial(
        pl.kernel,
        out_type=(
            jax.ShapeDtypeStruct((nb, dim, B), jnp.float32),
            jax.ShapeDtypeStruct((n,), jnp.float32),
        ),
        mesh=mesh,
        compiler_params=pltpu.CompilerParams(
            needs_layout_passes=False, use_tc_tiling_on_sc=False),
        scratch_types=[
            pltpu.VMEM((dim * (ninc + 1),), jnp.float32),
            pltpu.VMEM((dim * (ninc + 1),), jnp.int32),
            pltpu.VMEM((cb, dim, B), jnp.float32),
            pltpu.VMEM((cb, dim, B), jnp.float32),
            pltpu.VMEM((cb, dim, B), jnp.float32),
            pltpu.VMEM((cb, dim, B), jnp.float32),
            pltpu.VMEM((cb * B,), jnp.float32),
            pltpu.VMEM((cb * B,), jnp.float32),
            pltpu.SemaphoreType.DMA,
            pltpu.SemaphoreType.DMA,
            pltpu.SemaphoreType.DMA,
            pltpu.SemaphoreType.DMA,
            pltpu.SemaphoreType.DMA,
            pltpu.SemaphoreType.DMA,
        ],
    )
    def sc_kernel(y_hbm, grid_hbm, pair_hbm, x_hbm, lj_hbm,
                  grid_v, pair_v, ybuf0, ybuf1, xbuf0, xbuf1,
                  ljbuf0, ljbuf1, ysem0, ysem1, xsem0, xsem1,
                  ljsem0, ljsem1):
        wid = lax.axis_index("s") * nc + lax.axis_index("c")
        base_blk = wid * blocks_w
        ybufs, xbufs, ljbufs = (ybuf0, ybuf1), (xbuf0, xbuf1), (ljbuf0, ljbuf1)
        ysems, xsems, ljsems = (ysem0, ysem1), (xsem0, xsem1), (ljsem0, ljsem1)

        fninc = jnp.float32(ninc)

        def y_copy(c, b):
            blk0 = base_blk + c * cb
            return pltpu.make_async_copy(
                y_hbm.at[pl.ds(blk0, cb), :, :], ybufs[b], ysems[b])

        def x_copy(c, b):
            blk0 = base_blk + c * cb
            return pltpu.make_async_copy(
                xbufs[b], x_hbm.at[pl.ds(blk0, cb), :, :], xsems[b])

        def lj_copy(c, b):
            blk0 = base_blk + c * cb
            return pltpu.make_async_copy(
                ljbufs[b], lj_hbm.at[pl.ds(blk0 * B, cb * B)], ljsems[b])

        def compute_chunk(b):
            ybuf, xbuf, ljbuf = ybufs[b], xbufs[b], ljbufs[b]

            def group_body(g):
                bb = lax.shift_right_logical(g, jpb_bits)
                j0 = lax.shift_left(g & (jpb - 1), 4)
                lj = jnp.zeros((L,), jnp.float32)
                for d in range(dim):
                    doff = jnp.full((L,), d * (ninc + 1), jnp.int32)
                    yv = ybuf[bb, d, pl.ds(j0, L)]
                    t = yv * fninc
                    iy = t.astype(jnp.int32)
                    dy = t - iy.astype(jnp.float32)
                    idx = iy + doff
                    g_at = plsc.load_gather(grid_v, [idx])
                    w = plsc.load_gather(pair_v, [idx])
                    i_at = plsc.bitcast(w, jnp.float32)
                    l_at = plsc.bitcast(lax.shift_left(w, 16), jnp.float32)
                    xbuf[bb, d, pl.ds(j0, L)] = g_at + i_at * dy
                    lj = lj + l_at
                ljbuf[pl.ds(lax.shift_left(bb, 7) + j0, L)] = lj

            plsc.parallel_loop(0, cb * jpb, unroll=4)(group_body)

        y_copy(0, 0).start()
        pltpu.sync_copy(grid_hbm, grid_v)
        pltpu.sync_copy(pair_hbm, pair_v)

        def super_body(s, carry):
            for b in range(2):
                c = 2 * s + b
                y_copy(c, b).wait()
                nxt = c + 1

                @pl.when(nxt < 2 * n_super)
                def _():
                    y_copy(nxt, 1 - b).start()

                @pl.when(s > 0)
                def _():
                    x_copy(c - 2, b).wait()
                    lj_copy(c - 2, b).wait()

                compute_chunk(b)
                x_copy(c, b).start()
                lj_copy(c, b).start()
            return carry

        lax.fori_loop(0, n_super, super_body, 0, unroll=False)
        for b in range(2):
            x_copy(2 * n_super - 2 + b, b).wait()
            lj_copy(2 * n_super - 2 + b, b).wait()

    return sc_kernel


def kernel(y, grid, inc):
    n, dim = y.shape
    ninc = inc.shape[1]
    inc_p = jnp.concatenate([inc, inc[:, -1:]], axis=1)
    linc = jnp.log(inc_p * jnp.float32(ninc))
    hi = lax.bitcast_convert_type(inc_p.astype(jnp.bfloat16), jnp.uint16)
    lo = lax.bitcast_convert_type(linc.astype(jnp.bfloat16), jnp.uint16)
    pair = lax.bitcast_convert_type(
        (hi.astype(jnp.uint32) << 16) | lo.astype(jnp.uint32), jnp.int32)
    y3 = y.reshape(n // B, B, dim).transpose(0, 2, 1)
    sc_call = _make_sc_call(n, dim, ninc)
    x3, lj = sc_call(y3, grid.reshape(-1), pair.reshape(-1))
    x = x3.transpose(0, 2, 1).reshape(n, dim)
    return x, lj


# speedup vs baseline: 1.0197x; 1.0197x over previous
import functools

import jax
import jax.numpy as jnp
from jax import lax
from jax.experimental import pallas as pl
from jax.experimental.pallas import tpu as pltpu
from jax.experimental.pallas import tpu_sc as plsc

L = 16
B = 128


def _make_sc_call(n, dim, ninc):
    info = plsc.get_sparse_core_info()
    nc, ns = info.num_cores, info.num_subcores
    nw = nc * ns
    nb = n // B
    assert n % (B * nw) == 0
    blocks_w = nb // nw
    cb = min(16, blocks_w)
    assert blocks_w % (2 * cb) == 0
    n_super = blocks_w // (2 * cb)
    jpb = B // L
    jpb_bits = jpb.bit_length() - 1
    assert jpb == 1 << jpb_bits and B == 128

    mesh = plsc.VectorSubcoreMesh(core_axis_name="c", subcore_axis_name="s")

    @functools.partial(
        pl.kernel,
        out_type=(
            jax.ShapeDtypeStruct((nb, dim, B), jnp.float32),
            jax.ShapeDtypeStruct((n,), jnp.float32),
        ),
        mesh=mesh,
        compiler_params=pltpu.CompilerParams(
            needs_layout_passes=False, use_tc_tiling_on_sc=False),
        scratch_types=[
            pltpu.VMEM((dim * (ninc + 1),), jnp.float32),
            pltpu.VMEM((dim * (ninc + 1),), jnp.int32),
            pltpu.VMEM((cb, dim, B), jnp.float32),
            pltpu.VMEM((cb, dim, B), jnp.float32),
            pltpu.VMEM((cb, dim, B), jnp.float32),
            pltpu.VMEM((cb, dim, B), jnp.float32),
            pltpu.VMEM((cb * B,), jnp.float32),
            pltpu.VMEM((cb * B,), jnp.float32),
            pltpu.SemaphoreType.DMA,
            pltpu.SemaphoreType.DMA,
            pltpu.SemaphoreType.DMA,
            pltpu.SemaphoreType.DMA,
            pltpu.SemaphoreType.DMA,
            pltpu.SemaphoreType.DMA,
        ],
    )
    def sc_kernel(y_hbm, grid_hbm, pair_hbm, x_hbm, lj_hbm,
                  grid_v, pair_v, ybuf0, ybuf1, xbuf0, xbuf1,
                  ljbuf0, ljbuf1, ysem0, ysem1, xsem0, xsem1,
                  ljsem0, ljsem1):
        wid = lax.axis_index("s") * nc + lax.axis_index("c")
        base_blk = wid * blocks_w
        ybufs, xbufs, ljbufs = (ybuf0, ybuf1), (xbuf0, xbuf1), (ljbuf0, ljbuf1)
        ysems, xsems, ljsems = (ysem0, ysem1), (xsem0, xsem1), (ljsem0, ljsem1)

        fninc = jnp.float32(ninc)

        def y_copy(c, b):
            blk0 = base_blk + c * cb
            return pltpu.make_async_copy(
                y_hbm.at[pl.ds(blk0, cb), :, :], ybufs[b], ysems[b])

        def x_copy(c, b):
            blk0 = base_blk + c * cb
            return pltpu.make_async_copy(
                xbufs[b], x_hbm.at[pl.ds(blk0, cb), :, :], xsems[b])

        def lj_copy(c, b):
            blk0 = base_blk + c * cb
            return pltpu.make_async_copy(
                ljbufs[b], lj_hbm.at[pl.ds(blk0 * B, cb * B)], ljsems[b])

        def compute_chunk(b):
            ybuf, xbuf, ljbuf = ybufs[b], xbufs[b], ljbufs[b]

            def group_body(g):
                bb = lax.shift_right_logical(g, jpb_bits)
                j0 = lax.shift_left(g & (jpb - 1), 4)
                lj = jnp.zeros((L,), jnp.float32)
                for d in range(dim):
                    doff = jnp.full((L,), d * (ninc + 1), jnp.int32)
                    yv = ybuf[bb, d, pl.ds(j0, L)]
                    t = yv * fninc
                    iy = t.astype(jnp.int32)
                    dy = t - iy.astype(jnp.float32)
                    idx = iy + doff
                    g_at = plsc.load_gather(grid_v, [idx])
                    w = plsc.load_gather(pair_v, [idx])
                    i_at = plsc.bitcast(w, jnp.float32)
                    l_at = plsc.bitcast(lax.shift_left(w, 16), jnp.float32)
                    xbuf[bb, d, pl.ds(j0, L)] = g_at + i_at * dy
                    lj = lj + l_at
                ljbuf[pl.ds(lax.shift_left(bb, 7) + j0, L)] = lj

            plsc.parallel_loop(0, cb * jpb, unroll=4)(group_body)

        y_copy(0, 0).start()
        pltpu.sync_copy(grid_hbm, grid_v)
        pltpu.sync_copy(pair_hbm, pair_v)

        def super_body(s, carry):
            for b in range(2):
                c = 2 * s + b
                y_copy(c, b).wait()
                nxt = c + 1

                @pl.when(nxt < 2 * n_super)
                def _():
                    y_copy(nxt, 1 - b).start()

                @pl.when(s > 0)
                def _():
                    x_copy(c - 2, b).wait()
                    lj_copy(c - 2, b).wait()

                compute_chunk(b)
                x_copy(c, b).start()
                lj_copy(c, b).start()
            return carry

        lax.fori_loop(0, n_super, super_body, 0, unroll=False)
        for b in range(2):
            x_copy(2 * n_super - 2 + b, b).wait()
            lj_copy(2 * n_super - 2 + b, b).wait()

    return sc_kernel


def kernel(y, grid, inc):
    n, dim = y.shape
    ninc = inc.shape[1]
    inc_p = jnp.concatenate([inc, inc[:, -1:]], axis=1)
    linc = jnp.log(inc_p * jnp.float32(ninc))
    hi = lax.bitcast_convert_type(inc_p.astype(jnp.bfloat16), jnp.uint16)
    lo = lax.bitcast_convert_type(linc.astype(jnp.bfloat16), jnp.uint16)
    pair = lax.bitcast_convert_type(
        (hi.astype(jnp.uint32) << 16) | lo.astype(jnp.uint32), jnp.int32)
    y3 = y.reshape(n // B, B, dim).transpose(0, 2, 1)
    sc_call = _make_sc_call(n, dim, ninc)
    x3, lj = sc_call(y3, grid.reshape(-1), pair.reshape(-1))
    x = x3.transpose(0, 2, 1).reshape(n, dim)
    return x, lj
